# Initial kernel scaffold; baseline (speedup 1.0000x reference)
import jax
import jax.numpy as jnp
from jax.experimental import pallas as pl


def kernel(x, edge_index, edge_weight, W):
    raise NotImplementedError("write your pallas kernel here")


# speedup vs baseline: 2.7824x; 2.7824x over previous
import functools

import jax
import jax.numpy as jnp
from jax import lax
from jax.experimental import pallas as pl
from jax.experimental.pallas import tpu as pltpu
from jax.experimental.pallas import tpu_sc as plsc

N = 10000
E = 160000
D_IN = 256
D_OUT = 256
H = 128
NC = 2
NS = 16
LANES = 16
CHUNK = 128
NCH = 80
EPT = NCH * CHUNK
E_PAD = NS * EPT
RPT = 624
TAIL = N - NS * RPT
MBLK = 1000


def _matmul_body(x_ref, w_ref, o_ref):
    o_ref[...] = lax.dot_general(
        x_ref[...], w_ref[...],
        dimension_numbers=(((1,), (1,)), ((), ())),
        preferred_element_type=jnp.float32,
    )


def _support_halves(x, W):
    return pl.pallas_call(
        _matmul_body,
        grid=(NC, N // MBLK),
        in_specs=[
            pl.BlockSpec((MBLK, D_IN), lambda h, i: (i, 0)),
            pl.BlockSpec((H, D_IN), lambda h, i: (h, 0)),
        ],
        out_specs=pl.BlockSpec((MBLK, H), lambda h, i: (h * (N // MBLK) + i, 0)),
        out_shape=jax.ShapeDtypeStruct((NC * N, H), jnp.float32),
    )(x, W)


def _lane_bcast(v16, lane):
    idx = jnp.full((LANES, 1), lane, dtype=jnp.int32)
    return lax.gather(
        v16, idx,
        lax.GatherDimensionNumbers(
            offset_dims=(), collapsed_slice_dims=(0,), start_index_map=(0,)),
        slice_sizes=(1,),
        mode=lax.GatherScatterMode.PROMISE_IN_BOUNDS,
    )


def _aggregate(support2, row3, col3, w3):
    mesh = plsc.VectorSubcoreMesh(core_axis_name="c", subcore_axis_name="s")

    @functools.partial(
        pl.kernel,
        out_type=jax.ShapeDtypeStruct((NC * N, H), jnp.float32),
        mesh=mesh,
        scratch_types=[
            pltpu.VMEM_SHARED((N, H), jnp.float32),
            pltpu.VMEM((NCH, CHUNK), jnp.int32),
            pltpu.VMEM((NCH, CHUNK), jnp.int32),
            pltpu.VMEM((NCH, CHUNK), jnp.float32),
            pltpu.VMEM((CHUNK, H), jnp.float32),
        ],
    )
    def kern(sup_hbm, row_hbm, col_hbm, w_hbm, out_hbm,
             acc, colv, rowv, wv, rows):
        c = lax.axis_index("c")
        s = lax.axis_index("s")

        @pl.loop(0, CHUNK)
        def _(i):
            @pl.loop(0, H // LANES)
            def _(j):
                rows[i, pl.ds(j * LANES, LANES)] = jnp.zeros((LANES,), jnp.float32)

        @pl.loop(0, RPT // CHUNK)
        def _(z):
            pltpu.sync_copy(rows, acc.at[pl.ds(s * RPT + z * CHUNK, CHUNK)])

        pltpu.sync_copy(rows.at[pl.ds(0, RPT % CHUNK)],
                        acc.at[pl.ds(s * RPT + RPT - RPT % CHUNK, RPT % CHUNK)])

        @pl.when(s == NS - 1)
        def _():
            pltpu.sync_copy(rows.at[pl.ds(0, TAIL)], acc.at[pl.ds(NS * RPT, TAIL)])

        pltpu.sync_copy(col_hbm.at[s], colv)
        pltpu.sync_copy(row_hbm.at[s], rowv)
        pltpu.sync_copy(w_hbm.at[s], wv)

        base = c * N

        @pl.loop(0, NCH)
        def _(k):
            for g in range(CHUNK // LANES):
                sl = pl.ds(g * LANES, LANES)
                colv[k, sl] = colv[k, sl] + jnp.full((LANES,), base, jnp.int32)

        plsc.subcore_barrier()

        @pl.loop(0, NCH)
        def _(k):
            pltpu.sync_copy(sup_hbm.at[colv.at[k]], rows)
            for g in range(CHUNK // LANES):
                w16 = wv[k, pl.ds(g * LANES, LANES)]

                @pl.loop(0, LANES)
                def _(e, g=g, w16=w16):
                    wb = _lane_bcast(w16, e)
                    eidx = g * LANES + e
                    for j in range(H // LANES):
                        sl = pl.ds(j * LANES, LANES)
                        rows[eidx, sl] = rows[eidx, sl] * wb

            pltpu.sync_copy(rows, acc.at[rowv.at[k]], add=True)

        plsc.subcore_barrier()

        pltpu.sync_copy(acc.at[pl.ds(s * RPT, RPT)],
                        out_hbm.at[pl.ds(c * N + s * RPT, RPT)])

        @pl.when(s == NS - 1)
        def _():
            pltpu.sync_copy(acc.at[pl.ds(NS * RPT, TAIL)],
                            out_hbm.at[pl.ds(c * N + NS * RPT, TAIL)])

    return kern(support2, row3, col3, w3)


def kernel(x, edge_index, edge_weight, W):
    support2 = _support_halves(x, W)
    pad = E_PAD - E
    ipad = jnp.zeros((pad,), jnp.int32)
    row3 = jnp.concatenate([edge_index[0], ipad]).reshape(NS, NCH, CHUNK)
    col3 = jnp.concatenate([edge_index[1], ipad]).reshape(NS, NCH, CHUNK)
    w3 = jnp.concatenate([edge_weight, jnp.zeros((pad,), jnp.float32)]
                         ).reshape(NS, NCH, CHUNK)
    out2 = _aggregate(support2, row3, col3, w3)
    return out2.reshape(NC, N, H).transpose(1, 0, 2).reshape(N, D_OUT)


# speedup vs baseline: 3.4179x; 1.2284x over previous
import functools

import jax
import jax.numpy as jnp
from jax import lax
from jax.experimental import pallas as pl
from jax.experimental.pallas import tpu as pltpu
from jax.experimental.pallas import tpu_sc as plsc

N = 10000
E = 160000
D_IN = 256
D_OUT = 256
H = 128
NC = 2
NS = 16
LANES = 16
CHUNK = 128
NCH = 80
PH = NCH // 2
EPT = NCH * CHUNK
E_PAD = NS * EPT
RPT = 624
TAIL = N - NS * RPT
MBLK = 1000


def _matmul_body(x_ref, w_ref, o_ref):
    o_ref[...] = lax.dot_general(
        x_ref[...], w_ref[...],
        dimension_numbers=(((1,), (1,)), ((), ())),
        preferred_element_type=jnp.float32,
    )


def _support_halves(x, W):
    return pl.pallas_call(
        _matmul_body,
        grid=(NC, N // MBLK),
        in_specs=[
            pl.BlockSpec((MBLK, D_IN), lambda h, i: (i, 0)),
            pl.BlockSpec((H, D_IN), lambda h, i: (h, 0)),
        ],
        out_specs=pl.BlockSpec((MBLK, H), lambda h, i: (h * (N // MBLK) + i, 0)),
        out_shape=jax.ShapeDtypeStruct((NC * N, H), jnp.float32),
    )(x, W)


def _lane_bcast(v16, lane):
    idx = jnp.full((LANES, 1), lane, dtype=jnp.int32)
    return lax.gather(
        v16, idx,
        lax.GatherDimensionNumbers(
            offset_dims=(), collapsed_slice_dims=(0,), start_index_map=(0,)),
        slice_sizes=(1,),
        mode=lax.GatherScatterMode.PROMISE_IN_BOUNDS,
    )


def _aggregate(support2, row3, col3, w3):
    mesh = plsc.VectorSubcoreMesh(core_axis_name="c", subcore_axis_name="s")

    @functools.partial(
        pl.kernel,
        out_type=jax.ShapeDtypeStruct((NC * N, H), jnp.float32),
        mesh=mesh,
        scratch_types=[
            pltpu.VMEM_SHARED((N, H), jnp.float32),
            pltpu.VMEM((PH, CHUNK), jnp.int32),
            pltpu.VMEM((PH, CHUNK), jnp.int32),
            pltpu.VMEM((PH, CHUNK), jnp.float32),
            pltpu.VMEM((CHUNK, H), jnp.float32),
            pltpu.VMEM((CHUNK, H), jnp.float32),
            pltpu.SemaphoreType.DMA,
            pltpu.SemaphoreType.DMA,
            pltpu.SemaphoreType.DMA,
            pltpu.SemaphoreType.DMA,
        ],
    )
    def kern(sup_hbm, row_hbm, col_hbm, w_hbm, out_hbm,
             acc, colv, rowv, wv, rows0, rows1, sg0, sg1, ss0, ss1):
        c = lax.axis_index("c")
        s = lax.axis_index("s")

        @pl.loop(0, CHUNK)
        def _(i):
            @pl.loop(0, H // LANES)
            def _(j):
                rows0[i, pl.ds(j * LANES, LANES)] = jnp.zeros((LANES,), jnp.float32)

        @pl.loop(0, RPT // CHUNK)
        def _(z):
            pltpu.sync_copy(rows0, acc.at[pl.ds(s * RPT + z * CHUNK, CHUNK)])

        pltpu.sync_copy(rows0.at[pl.ds(0, RPT % CHUNK)],
                        acc.at[pl.ds(s * RPT + RPT - RPT % CHUNK, RPT % CHUNK)])

        @pl.when(s == NS - 1)
        def _():
            pltpu.sync_copy(rows0.at[pl.ds(0, TAIL)], acc.at[pl.ds(NS * RPT, TAIL)])

        base = c * N

        def gather_start(kk, buf, sem):
            pltpu.async_copy(sup_hbm.at[colv.at[kk]], buf, sem)

        def gather_wait(kk, buf, sem):
            pltpu.make_async_copy(sup_hbm.at[colv.at[kk]], buf, sem).wait()

        def scat_start(kk, buf, sem):
            pltpu.async_copy(buf, acc.at[rowv.at[kk]], sem, add=True)

        def scat_wait(kk, buf, sem):
            pltpu.make_async_copy(buf, acc.at[rowv.at[kk]], sem).wait()

        def scale(kk, buf):
            for g in range(CHUNK // LANES):
                w16 = wv[kk, pl.ds(g * LANES, LANES)]

                @pl.loop(0, LANES, unroll=4)
                def _(e, g=g, w16=w16, buf=buf):
                    wb = _lane_bcast(w16, e)
                    eidx = g * LANES + e
                    for j in range(H // LANES):
                        sl = pl.ds(j * LANES, LANES)
                        buf[eidx, sl] = buf[eidx, sl] * wb

        plsc.subcore_barrier()

        for p in range(NCH // PH):
            psl = pl.ds(p * PH, PH)
            pltpu.sync_copy(col_hbm.at[s, psl], colv)
            pltpu.sync_copy(row_hbm.at[s, psl], rowv)
            pltpu.sync_copy(w_hbm.at[s, psl], wv)

            @pl.loop(0, PH)
            def _(k):
                for g in range(CHUNK // LANES):
                    sl = pl.ds(g * LANES, LANES)
                    colv[k, sl] = colv[k, sl] + jnp.full((LANES,), base, jnp.int32)

            gather_start(0, rows0, sg0)

            @pl.loop(0, PH, step=2)
            def _(k):
                @pl.when(k > 0)
                def _():
                    scat_wait(k - 1, rows1, ss1)

                gather_start(k + 1, rows1, sg1)
                gather_wait(k, rows0, sg0)
                scale(k, rows0)
                scat_start(k, rows0, ss0)

                @pl.when(k + 2 < PH)
                def _():
                    scat_wait(k, rows0, ss0)
                    gather_start(k + 2, rows0, sg0)

                gather_wait(k + 1, rows1, sg1)
                scale(k + 1, rows1)
                scat_start(k + 1, rows1, ss1)

            scat_wait(PH - 2, rows0, ss0)
            scat_wait(PH - 1, rows1, ss1)

        plsc.subcore_barrier()

        pltpu.sync_copy(acc.at[pl.ds(s * RPT, RPT)],
                        out_hbm.at[pl.ds(c * N + s * RPT, RPT)])

        @pl.when(s == NS - 1)
        def _():
            pltpu.sync_copy(acc.at[pl.ds(NS * RPT, TAIL)],
                            out_hbm.at[pl.ds(c * N + NS * RPT, TAIL)])

    return kern(support2, row3, col3, w3)


def kernel(x, edge_index, edge_weight, W):
    support2 = _support_halves(x, W)
    pad = E_PAD - E
    ipad = jnp.zeros((pad,), jnp.int32)
    row3 = jnp.concatenate([edge_index[0], ipad]).reshape(NS, NCH, CHUNK)
    col3 = jnp.concatenate([edge_index[1], ipad]).reshape(NS, NCH, CHUNK)
    w3 = jnp.concatenate([edge_weight, jnp.zeros((pad,), jnp.float32)]
                         ).reshape(NS, NCH, CHUNK)
    out2 = _aggregate(support2, row3, col3, w3)
    return out2.reshape(NC, N, H).transpose(1, 0, 2).reshape(N, D_OUT)
